# Initial kernel scaffold; baseline (speedup 1.0000x reference)
import jax
import jax.numpy as jnp
from jax.experimental import pallas as pl


def kernel(features, edge_index, W1, b1, W2, b2):
    raise NotImplementedError("write your pallas kernel here")


# speedup vs baseline: 15.5517x; 15.5517x over previous
import functools

import jax
import jax.numpy as jnp
from jax import lax
from jax.experimental import pallas as pl
from jax.experimental.pallas import tpu as pltpu
from jax.experimental.pallas import tpu_sc as plsc

N_NODES = 10000
D_FEAT = 128
D_HID = 64
NC, NS = 2, 16
NW = NC * NS
N_PAD = 10240
RPT = N_PAD // NS
CHUNK = 128

MB = 512
GRID_M = N_PAD // MB


def _sc_degree(dst_pad, zeros1):
    e_pad = dst_pad.shape[0]
    ept = e_pad // NW
    n_chunks = ept // CHUNK
    mesh = plsc.VectorSubcoreMesh(core_axis_name="c", subcore_axis_name="s")

    @functools.partial(
        pl.kernel,
        out_type=jax.ShapeDtypeStruct((NC, N_PAD), jnp.float32),
        mesh=mesh,
        scratch_types=[
            pltpu.VMEM((CHUNK,), jnp.int32),
            pltpu.VMEM((CHUNK,), jnp.float32),
            pltpu.VMEM_SHARED((N_PAD,), jnp.float32),
        ],
    )
    def deg_kernel(dst_hbm, zeros_hbm, out_hbm, idx_v, ones_v, deg_sh):
        c = lax.axis_index("c")
        s = lax.axis_index("s")
        base = (c * NS + s) * ept
        r0 = s * RPT
        pltpu.sync_copy(zeros_hbm.at[pl.ds(r0, RPT)], deg_sh.at[pl.ds(r0, RPT)])
        for i in range(CHUNK // 16):
            ones_v[pl.ds(i * 16, 16)] = jnp.ones((16,), jnp.float32)
        plsc.subcore_barrier()

        def body(k, carry):
            off = base + k * CHUNK
            pltpu.sync_copy(dst_hbm.at[pl.ds(off, CHUNK)], idx_v)
            pltpu.sync_copy(ones_v, deg_sh.at[idx_v], add=True)
            return carry

        lax.fori_loop(0, n_chunks, body, 0)
        plsc.subcore_barrier()
        pltpu.sync_copy(deg_sh.at[pl.ds(r0, RPT)],
                        out_hbm.at[c, pl.ds(r0, RPT)])

    return deg_kernel(dst_pad, zeros1)


def _sc_propagate(hs_pad, src_pad, dst_pad, zeros2):
    e_pad = src_pad.shape[0]
    ept = e_pad // NW
    n_chunks = ept // CHUNK
    mesh = plsc.VectorSubcoreMesh(core_axis_name="c", subcore_axis_name="s")

    @functools.partial(
        pl.kernel,
        out_type=jax.ShapeDtypeStruct((NC, N_PAD, D_HID), jnp.float32),
        mesh=mesh,
        scratch_types=[
            pltpu.VMEM((CHUNK,), jnp.int32),
            pltpu.VMEM((CHUNK,), jnp.int32),
            pltpu.VMEM((CHUNK, D_HID), jnp.float32),
            pltpu.VMEM_SHARED((N_PAD, D_HID), jnp.float32),
            pltpu.SemaphoreType.DMA,
        ],
        compiler_params=pltpu.CompilerParams(use_tc_tiling_on_sc=False),
    )
    def prop_kernel(hs_hbm, src_hbm, dst_hbm, zeros_hbm, out_hbm,
                    sidx, didx, rows, acc_sh, sem):
        c = lax.axis_index("c")
        s = lax.axis_index("s")
        base = (c * NS + s) * ept
        r0 = s * RPT
        pltpu.sync_copy(zeros_hbm.at[pl.ds(r0, RPT)], acc_sh.at[pl.ds(r0, RPT)])
        plsc.subcore_barrier()

        def body(k, carry):
            off = base + k * CHUNK
            pltpu.sync_copy(src_hbm.at[pl.ds(off, CHUNK)], sidx)
            pltpu.sync_copy(dst_hbm.at[pl.ds(off, CHUNK)], didx)
            pltpu.async_copy(hs_hbm.at[sidx], rows, sem).wait()
            pltpu.sync_copy(rows, acc_sh.at[didx], add=True)
            return carry

        lax.fori_loop(0, n_chunks, body, 0)
        plsc.subcore_barrier()
        pltpu.sync_copy(acc_sh.at[pl.ds(r0, RPT)],
                        out_hbm.at[c, pl.ds(r0, RPT)])

    return prop_kernel(hs_pad, src_pad, dst_pad, zeros2)


def _tc_first(deg_parts_t, x_pad, W1):
    def body(deg_ref, x_ref, w_ref, hs_ref, dinv_ref):
        deg = deg_ref[...]
        degt = deg[:, 0:1] + deg[:, 1:2] + 1.0
        dinv = lax.rsqrt(jnp.maximum(degt, 1e-12))
        h = jnp.dot(x_ref[...], w_ref[...], preferred_element_type=jnp.float32)
        hs_ref[...] = dinv * h
        dinv_ref[...] = dinv

    return pl.pallas_call(
        body,
        grid=(GRID_M,),
        in_specs=[
            pl.BlockSpec((MB, NC), lambda i: (i, 0)),
            pl.BlockSpec((MB, D_FEAT), lambda i: (i, 0)),
            pl.BlockSpec((D_FEAT, D_HID), lambda i: (0, 0)),
        ],
        out_specs=[
            pl.BlockSpec((MB, D_HID), lambda i: (i, 0)),
            pl.BlockSpec((MB, 1), lambda i: (i, 0)),
        ],
        out_shape=[
            jax.ShapeDtypeStruct((N_PAD, D_HID), jnp.float32),
            jax.ShapeDtypeStruct((N_PAD, 1), jnp.float32),
        ],
    )(deg_parts_t, x_pad, W1)


def _tc_mid(acc_parts, hs1, dinv, b1, W2):
    def body(acc_ref, hs_ref, dinv_ref, b_ref, w_ref, out_ref):
        agg = acc_ref[0] + acc_ref[1] + hs_ref[...]
        pre = dinv_ref[...] * agg + b_ref[...]
        h = jnp.maximum(pre, 0.0)
        h2 = jnp.dot(h, w_ref[...], preferred_element_type=jnp.float32)
        out_ref[...] = dinv_ref[...] * h2

    return pl.pallas_call(
        body,
        grid=(GRID_M,),
        in_specs=[
            pl.BlockSpec((NC, MB, D_HID), lambda i: (0, i, 0)),
            pl.BlockSpec((MB, D_HID), lambda i: (i, 0)),
            pl.BlockSpec((MB, 1), lambda i: (i, 0)),
            pl.BlockSpec((1, D_HID), lambda i: (0, 0)),
            pl.BlockSpec((D_HID, D_HID), lambda i: (0, 0)),
        ],
        out_specs=pl.BlockSpec((MB, D_HID), lambda i: (i, 0)),
        out_shape=jax.ShapeDtypeStruct((N_PAD, D_HID), jnp.float32),
    )(acc_parts, hs1, dinv, b1, W2)


def _tc_final(acc_parts, hs2, dinv, b2):
    def body(acc_ref, hs_ref, dinv_ref, b_ref, out_ref):
        agg = acc_ref[0] + acc_ref[1] + hs_ref[...]
        out_ref[...] = dinv_ref[...] * agg + b_ref[...]

    return pl.pallas_call(
        body,
        grid=(GRID_M,),
        in_specs=[
            pl.BlockSpec((NC, MB, D_HID), lambda i: (0, i, 0)),
            pl.BlockSpec((MB, D_HID), lambda i: (i, 0)),
            pl.BlockSpec((MB, 1), lambda i: (i, 0)),
            pl.BlockSpec((1, D_HID), lambda i: (0, 0)),
        ],
        out_specs=pl.BlockSpec((MB, D_HID), lambda i: (i, 0)),
        out_shape=jax.ShapeDtypeStruct((N_PAD, D_HID), jnp.float32),
    )(acc_parts, hs2, dinv, b2)


def kernel(features, edge_index, W1, b1, W2, b2):
    src = edge_index[0].astype(jnp.int32)
    dst = edge_index[1].astype(jnp.int32)
    e = src.shape[0]
    quantum = NW * CHUNK
    e_pad = ((e + quantum - 1) // quantum) * quantum
    fill = jnp.full((e_pad - e,), N_NODES, jnp.int32)
    src_p = jnp.concatenate([src, fill])
    dst_p = jnp.concatenate([dst, fill])

    x_pad = jnp.pad(features, ((0, N_PAD - N_NODES), (0, 0)))
    zeros1 = jnp.zeros((N_PAD,), jnp.float32)
    zeros2 = jnp.zeros((N_PAD, D_HID), jnp.float32)
    b1r = b1.reshape(1, D_HID)
    b2r = b2.reshape(1, D_HID)

    deg_parts = _sc_degree(dst_p, zeros1)
    hs1, dinv = _tc_first(deg_parts.T, x_pad, W1)
    acc1 = _sc_propagate(hs1, src_p, dst_p, zeros2)
    hs2 = _tc_mid(acc1, hs1, dinv, b1r, W2)
    acc2 = _sc_propagate(hs2, src_p, dst_p, zeros2)
    out = _tc_final(acc2, hs2, dinv, b2r)
    return out[:N_NODES]
